# baseline (speedup 1.0000x reference)
import functools

import jax
import jax.numpy as jnp
from jax import lax
from jax.experimental import pallas as pl
from jax.experimental.pallas import tpu as pltpu
from jax.experimental.pallas import tpu_sc as plsc

_D = 64
_START = 400001
_END = 400002

_NC, _NS = 2, 16
_NW = _NC * _NS
_CHUNK = 128
_LANES = 16


def _body(n_chunks, idx_hbm, table_hbm, marker_hbm, out_hbm,
          idx_v, rows_v, marker_v):
  wid = lax.axis_index("s") * _NC + lax.axis_index("c")
  chunk0 = wid * n_chunks

  pltpu.sync_copy(idx_hbm.at[pl.ds(chunk0, n_chunks)], idx_v)
  pltpu.sync_copy(marker_hbm, marker_v)
  m0 = [marker_v[0, pl.ds(k * _LANES, _LANES)] for k in range(_D // _LANES)]
  m1 = [marker_v[1, pl.ds(k * _LANES, _LANES)] for k in range(_D // _LANES)]

  @pl.loop(0, n_chunks)
  def _chunk(j):
    pltpu.sync_copy(table_hbm.at[idx_v.at[j]], rows_v)

    acc = None
    for g in range(_CHUNK // _LANES):
      vg = idx_v[j, pl.ds(g * _LANES, _LANES)]
      mg = (vg == _START) | (vg == _END)
      acc = mg if acc is None else (acc | mg)
    any_hit = plsc.all_reduce_population_count(acc)[0] > 0

    @pl.when(any_hit)
    def _fix():
      @pl.loop(0, _CHUNK // _LANES)
      def _grp(g):
        vg = idx_v[j, pl.ds(g * _LANES, _LANES)]
        for r in range(_LANES):
          s = vg[r]
          row = g * _LANES + r

          @pl.when(s == _START)
          def _():
            for k in range(_D // _LANES):
              rows_v[row, pl.ds(k * _LANES, _LANES)] = m0[k]

          @pl.when(s == _END)
          def _():
            for k in range(_D // _LANES):
              rows_v[row, pl.ds(k * _LANES, _LANES)] = m1[k]

    pltpu.sync_copy(rows_v, out_hbm.at[pl.ds((chunk0 + j) * _CHUNK, _CHUNK)])


def kernel(idxes, embeddings_weight, marker_weight):
  n_rows = idxes.size
  n_chunks = n_rows // (_NW * _CHUNK)
  assert n_rows == _NW * _CHUNK * n_chunks
  idx_flat = idxes.reshape(_NW * n_chunks, _CHUNK)

  run = pl.kernel(
      functools.partial(_body, n_chunks),
      out_type=jax.ShapeDtypeStruct((n_rows, _D), jnp.float32),
      mesh=plsc.VectorSubcoreMesh(core_axis_name="c", subcore_axis_name="s"),
      compiler_params=pltpu.CompilerParams(
          needs_layout_passes=False, use_tc_tiling_on_sc=False),
      scratch_types=[
          pltpu.VMEM((n_chunks, _CHUNK), jnp.int32),
          pltpu.VMEM((_CHUNK, _D), jnp.float32),
          pltpu.VMEM((2, _D), jnp.float32),
      ],
  )
  out = run(idx_flat, embeddings_weight, marker_weight)
  return out.reshape(idxes.shape + (_D,))


# speedup vs baseline: 1.1688x; 1.1688x over previous
import functools

import jax
import jax.numpy as jnp
from jax import lax
from jax.experimental import pallas as pl
from jax.experimental.pallas import tpu as pltpu
from jax.experimental.pallas import tpu_sc as plsc

_D = 64
_START = 400001
_END = 400002

_NC, _NS = 2, 16
_NW = _NC * _NS
_CHUNK = 128
_LANES = 16
_NBUF = 4


def _body(n_chunks, idx_hbm, table_hbm, marker_hbm, out_hbm,
          idx_v, marker_v, rows, sem_g, sem_w):
  wid = lax.axis_index("s") * _NC + lax.axis_index("c")
  chunk0 = wid * n_chunks

  pltpu.sync_copy(idx_hbm.at[pl.ds(chunk0, n_chunks)], idx_v)
  pltpu.sync_copy(marker_hbm, marker_v)
  m0 = [marker_v[0, pl.ds(k * _LANES, _LANES)] for k in range(_D // _LANES)]
  m1 = [marker_v[1, pl.ds(k * _LANES, _LANES)] for k in range(_D // _LANES)]

  def fire_gather(b, j):
    pltpu.async_copy(table_hbm.at[idx_v.at[j]], rows[b], sem_g[b])

  def wait_gather(b, j):
    pltpu.make_async_copy(table_hbm.at[idx_v.at[j]], rows[b], sem_g[b]).wait()

  def fire_write(b, j):
    pltpu.async_copy(
        rows[b], out_hbm.at[pl.ds((chunk0 + j) * _CHUNK, _CHUNK)], sem_w[b])

  def wait_write(b, j):
    pltpu.make_async_copy(
        rows[b], out_hbm.at[pl.ds((chunk0 + j) * _CHUNK, _CHUNK)],
        sem_w[b]).wait()

  def fix_markers(b, j):
    acc = None
    for g in range(_CHUNK // _LANES):
      vg = idx_v[j, pl.ds(g * _LANES, _LANES)]
      mg = (vg == _START) | (vg == _END)
      acc = mg if acc is None else (acc | mg)
    any_hit = plsc.all_reduce_population_count(acc)[0] > 0

    @pl.when(any_hit)
    def _fix():
      @pl.loop(0, _CHUNK // _LANES)
      def _grp(g):
        vg = idx_v[j, pl.ds(g * _LANES, _LANES)]
        for r in range(_LANES):
          s = vg[r]
          row = g * _LANES + r

          @pl.when(s == _START)
          def _():
            for k in range(_D // _LANES):
              rows[b][row, pl.ds(k * _LANES, _LANES)] = m0[k]

          @pl.when(s == _END)
          def _():
            for k in range(_D // _LANES):
              rows[b][row, pl.ds(k * _LANES, _LANES)] = m1[k]

  for b in range(_NBUF):
    fire_gather(b, b)

  @pl.loop(0, n_chunks // _NBUF)
  def _super(js):
    for bi in range(_NBUF):
      j = js * _NBUF + bi

      bp = (bi - 1) % _NBUF
      can_prefetch = jnp.logical_and(j >= 1, j <= n_chunks - _NBUF)

      @pl.when(can_prefetch)
      def _prefetch():
        wait_write(bp, j - 1)
        fire_gather(bp, j + _NBUF - 1)

      wait_gather(bi, j)
      fix_markers(bi, j)
      fire_write(bi, j)

  for b in range(_NBUF):
    wait_write(b, n_chunks - _NBUF + b)


def kernel(idxes, embeddings_weight, marker_weight):
  n_rows = idxes.size
  n_chunks = n_rows // (_NW * _CHUNK)
  assert n_rows == _NW * _CHUNK * n_chunks and n_chunks % _NBUF == 0
  idx_flat = idxes.reshape(_NW * n_chunks, _CHUNK)

  run = pl.kernel(
      functools.partial(_body, n_chunks),
      out_type=jax.ShapeDtypeStruct((n_rows, _D), jnp.float32),
      mesh=plsc.VectorSubcoreMesh(core_axis_name="c", subcore_axis_name="s"),
      compiler_params=pltpu.CompilerParams(
          needs_layout_passes=False, use_tc_tiling_on_sc=False),
      scratch_types=[
          pltpu.VMEM((n_chunks, _CHUNK), jnp.int32),
          pltpu.VMEM((2, _D), jnp.float32),
          [pltpu.VMEM((_CHUNK, _D), jnp.float32) for _ in range(_NBUF)],
          [pltpu.SemaphoreType.DMA for _ in range(_NBUF)],
          [pltpu.SemaphoreType.DMA for _ in range(_NBUF)],
      ],
  )
  out = run(idx_flat, embeddings_weight, marker_weight)
  return out.reshape(idxes.shape + (_D,))


# speedup vs baseline: 1.4819x; 1.2679x over previous
import functools

import jax
import jax.numpy as jnp
from jax import lax
from jax.experimental import pallas as pl
from jax.experimental.pallas import tpu as pltpu
from jax.experimental.pallas import tpu_sc as plsc

_D = 64
_START = 400001
_END = 400002

_NC, _NS = 2, 16
_NW = _NC * _NS
_CHUNK = 128
_LANES = 16
_NBUF = 4
_DP = 128


def _body(n_chunks, idx_hbm, table_hbm, marker_hbm, out_hbm,
          idx_v, marker_v, rows, sem_g, sem_w):
  wid = lax.axis_index("s") * _NC + lax.axis_index("c")
  chunk0 = wid * n_chunks

  pltpu.sync_copy(idx_hbm.at[pl.ds(chunk0, n_chunks)], idx_v)
  pltpu.sync_copy(marker_hbm, marker_v)
  m0 = [marker_v[0, pl.ds(k * _LANES, _LANES)] for k in range(_D // _LANES)]
  m1 = [marker_v[1, pl.ds(k * _LANES, _LANES)] for k in range(_D // _LANES)]

  def fire_gather(b, j):
    pltpu.async_copy(table_hbm.at[idx_v.at[j]], rows[b], sem_g[b])

  def wait_gather(b, j):
    pltpu.make_async_copy(table_hbm.at[idx_v.at[j]], rows[b], sem_g[b]).wait()

  def out_dst(j):
    return out_hbm.at[pl.ds((chunk0 + j) * _CHUNK, _CHUNK)]

  def fire_write(b, j):
    pltpu.async_copy(rows[b], out_dst(j), sem_w[b])

  def wait_write(b, j):
    pltpu.make_async_copy(rows[b], out_dst(j), sem_w[b]).wait()

  def fix_markers(b, j):
    acc = None
    for g in range(_CHUNK // _LANES):
      vg = idx_v[j, pl.ds(g * _LANES, _LANES)]
      mg = (vg == _START) | (vg == _END)
      acc = mg if acc is None else (acc | mg)
    any_hit = plsc.all_reduce_population_count(acc)[0] > 0

    @pl.when(any_hit)
    def _fix():
      @pl.loop(0, _CHUNK // _LANES)
      def _grp(g):
        vg = idx_v[j, pl.ds(g * _LANES, _LANES)]
        for r in range(_LANES):
          s = vg[r]
          row = g * _LANES + r

          @pl.when(s == _START)
          def _():
            for k in range(_D // _LANES):
              rows[b][row, pl.ds(k * _LANES, _LANES)] = m0[k]

          @pl.when(s == _END)
          def _():
            for k in range(_D // _LANES):
              rows[b][row, pl.ds(k * _LANES, _LANES)] = m1[k]

  for b in range(_NBUF):
    fire_gather(b, b)

  @pl.loop(0, n_chunks // _NBUF)
  def _super(js):
    for bi in range(_NBUF):
      j = js * _NBUF + bi

      bp = (bi - 1) % _NBUF
      can_prefetch = jnp.logical_and(j >= 1, j <= n_chunks - _NBUF)

      @pl.when(can_prefetch)
      def _prefetch():
        wait_write(bp, j - 1)
        fire_gather(bp, j + _NBUF - 1)

      wait_gather(bi, j)
      fix_markers(bi, j)
      fire_write(bi, j)

  for b in range(_NBUF):
    wait_write(b, n_chunks - _NBUF + b)


def kernel(idxes, embeddings_weight, marker_weight):
  n_rows = idxes.size
  n_chunks = n_rows // (_NW * _CHUNK)
  assert n_rows == _NW * _CHUNK * n_chunks and n_chunks % _NBUF == 0
  idx_flat = idxes.reshape(_NW * n_chunks, _CHUNK)
  table_p = jnp.pad(embeddings_weight, ((0, 0), (0, _DP - _D)))
  marker_p = jnp.pad(marker_weight, ((0, 0), (0, _DP - _D)))

  run = pl.kernel(
      functools.partial(_body, n_chunks),
      out_type=jax.ShapeDtypeStruct((n_rows, _DP), jnp.float32),
      mesh=plsc.VectorSubcoreMesh(core_axis_name="c", subcore_axis_name="s"),
      compiler_params=pltpu.CompilerParams(
          needs_layout_passes=False, use_tc_tiling_on_sc=True),
      scratch_types=[
          pltpu.VMEM((n_chunks, _CHUNK), jnp.int32),
          pltpu.VMEM((2, _DP), jnp.float32),
          [pltpu.VMEM((_CHUNK, _DP), jnp.float32) for _ in range(_NBUF)],
          [pltpu.SemaphoreType.DMA for _ in range(_NBUF)],
          [pltpu.SemaphoreType.DMA for _ in range(_NBUF)],
      ],
  )
  out = run(idx_flat, table_p, marker_p)
  return out[:, :_D].reshape(idxes.shape + (_D,))
